# Initial kernel scaffold; baseline (speedup 1.0000x reference)
import jax
import jax.numpy as jnp
from jax import lax
from jax.experimental import pallas as pl
from jax.experimental.pallas import tpu as pltpu
from jax.experimental.pallas import tpu_sc as plsc

N_NODES = 10000
N_EDGES = 160000
D_IN = 256
D_OUT = 512
HALF = D_IN // 2

NC = 2
NS = 16
L = 16

EPT = N_EDGES // NS
K = 80
NCHUNK = EPT // K
NP = 10240
SLICE = NP // NS


def _rsqrt_pos(d):
    i = plsc.bitcast(d, jnp.int32)
    i = jnp.int32(0x5F3759DF) - lax.shift_right_logical(i, 1)
    y = plsc.bitcast(i, jnp.float32)
    half_d = 0.5 * d
    for _ in range(3):
        y = y * (1.5 - half_d * y * y)
    return y


def _sc_body(x2_hbm, row3_hbm, col3_hbm, ew3_hbm,
             agg_hbm, dis_hbm,
             agg_sp, deg_parts, dis_sh,
             row2d, col2d, ewn2d,
             deg_local, dis_local,
             part_buf, acc_buf, dis_buf,
             gbuf, gsem):
    c = lax.axis_index("c")
    s = lax.axis_index("s")
    base = s * SLICE
    zero16 = jnp.zeros((L,), jnp.float32)

    pltpu.sync_copy(row3_hbm.at[s], row2d)
    pltpu.sync_copy(col3_hbm.at[s], col2d)
    pltpu.sync_copy(ew3_hbm.at[s], ewn2d)

    def zero_deg(i, _):
        deg_local[pl.ds(i * L, L)] = zero16
        return 0
    lax.fori_loop(0, NP // L, zero_deg, 0)

    def deg_acc(j, _):
        for k in range(K // L):
            c16 = col2d[j, pl.ds(k * L, L)]
            w16 = ewn2d[j, pl.ds(k * L, L)]
            plsc.addupdate_scatter(deg_local, [c16], w16)
        return 0
    lax.fori_loop(0, NCHUNK, deg_acc, 0)

    def zero_g(e, _):
        for q in range(HALF // L):
            gbuf[e, pl.ds(q * L, L)] = zero16
        return 0
    lax.fori_loop(0, K, zero_g, 0)

    def zero_agg(t, _):
        pltpu.sync_copy(gbuf, agg_sp.at[pl.ds(base + t * K, K)])
        return 0
    lax.fori_loop(0, SLICE // K, zero_agg, 0)

    pltpu.sync_copy(deg_local, deg_parts.at[s])
    plsc.subcore_barrier()

    pltpu.sync_copy(deg_parts.at[0, pl.ds(base, SLICE)], acc_buf)

    def sum_part(p, _):
        pltpu.sync_copy(deg_parts.at[p, pl.ds(base, SLICE)], part_buf)
        def add_vec(i, _):
            acc_buf[pl.ds(i * L, L)] = (acc_buf[pl.ds(i * L, L)]
                                        + part_buf[pl.ds(i * L, L)])
            return 0
        lax.fori_loop(0, SLICE // L, add_vec, 0)
        return 0
    lax.fori_loop(1, NS, sum_part, 0)

    def calc_dis(i, _):
        d = acc_buf[pl.ds(i * L, L)] + 1.0
        dis_buf[pl.ds(i * L, L)] = _rsqrt_pos(d)
        return 0
    lax.fori_loop(0, SLICE // L, calc_dis, 0)

    pltpu.sync_copy(dis_buf, dis_sh.at[pl.ds(base, SLICE)])

    @pl.when(c == 0)
    def _():
        pltpu.sync_copy(dis_buf, dis_hbm.at[pl.ds(base, SLICE)])

    plsc.subcore_barrier()
    pltpu.sync_copy(dis_sh, dis_local)

    def norm_edges(j, _):
        for k in range(K // L):
            sl = pl.ds(k * L, L)
            r16 = row2d[j, sl]
            c16 = col2d[j, sl]
            dr = plsc.load_gather(dis_local, [r16])
            dc = plsc.load_gather(dis_local, [c16])
            ewn2d[j, sl] = dr * ewn2d[j, sl] * dc
        return 0
    lax.fori_loop(0, NCHUNK, norm_edges, 0)

    cN = c * N_NODES

    def offs(j, _):
        for k in range(K // L):
            sl = pl.ds(k * L, L)
            row2d[j, sl] = row2d[j, sl] + cN
        return 0
    lax.fori_loop(0, NCHUNK, offs, 0)

    def chunk(j, _):
        pltpu.async_copy(x2_hbm.at[row2d.at[j]], gbuf, gsem).wait()

        def scale(e, _):
            sv = ewn2d[j, e]
            for q in range(HALF // L):
                sl = pl.ds(q * L, L)
                gbuf[e, sl] = gbuf[e, sl] * sv
            return 0
        lax.fori_loop(0, K, scale, 0)

        pltpu.sync_copy(gbuf, agg_sp.at[col2d.at[j]], add=True)
        return 0
    lax.fori_loop(0, NCHUNK, chunk, 0)

    plsc.subcore_barrier()
    pltpu.sync_copy(agg_sp.at[pl.ds(base, SLICE)],
                    agg_hbm.at[pl.ds(c * NP + base, SLICE)])


def _sc_aggregate(x2, row3, col3, ew3):
    mesh = plsc.VectorSubcoreMesh(core_axis_name="c", subcore_axis_name="s",
                                  num_cores=NC, num_subcores=NS)
    return pl.kernel(
        _sc_body,
        out_type=(
            jax.ShapeDtypeStruct((NC * NP, HALF), jnp.float32),
            jax.ShapeDtypeStruct((NP,), jnp.float32),
        ),
        mesh=mesh,
        scratch_types=[
            pltpu.VMEM_SHARED((NP, HALF), jnp.float32),
            pltpu.VMEM_SHARED((NS, NP), jnp.float32),
            pltpu.VMEM_SHARED((NP,), jnp.float32),
            pltpu.VMEM((NCHUNK, K), jnp.int32),
            pltpu.VMEM((NCHUNK, K), jnp.int32),
            pltpu.VMEM((NCHUNK, K), jnp.float32),
            pltpu.VMEM((NP,), jnp.float32),
            pltpu.VMEM((NP,), jnp.float32),
            pltpu.VMEM((SLICE,), jnp.float32),
            pltpu.VMEM((SLICE,), jnp.float32),
            pltpu.VMEM((SLICE,), jnp.float32),
            pltpu.VMEM((K, HALF), jnp.float32),
            pltpu.SemaphoreType.DMA,
        ],
    )(x2, row3, col3, ew3)


_RBLK = 1000


def _tc_body(x_ref, agg_ref, dis_ref, w_ref, b_ref, out_ref):
    d = dis_ref[...]
    a = agg_ref[...] + (d * d) * x_ref[...]
    acc = jnp.dot(a, w_ref[...], preferred_element_type=jnp.float32)
    out_ref[...] = jnp.maximum(acc + b_ref[...], 0.0)


def _tc_finish(x, agg, dis2, W, b2):
    grid = (N_NODES // _RBLK,)
    return pl.pallas_call(
        _tc_body,
        grid=grid,
        in_specs=[
            pl.BlockSpec((_RBLK, D_IN), lambda i: (i, 0)),
            pl.BlockSpec((_RBLK, D_IN), lambda i: (i, 0)),
            pl.BlockSpec((_RBLK, 1), lambda i: (i, 0)),
            pl.BlockSpec((D_IN, D_OUT), lambda i: (0, 0)),
            pl.BlockSpec((1, D_OUT), lambda i: (0, 0)),
        ],
        out_specs=pl.BlockSpec((_RBLK, D_OUT), lambda i: (i, 0)),
        out_shape=jax.ShapeDtypeStruct((N_NODES, D_OUT), jnp.float32),
    )(x, agg, dis2, W, b2)


def kernel(x, edge_index, edge_weight, W, b):
    row = edge_index[0].astype(jnp.int32)
    col = edge_index[1].astype(jnp.int32)
    row3 = row.reshape(NS, NCHUNK, K)
    col3 = col.reshape(NS, NCHUNK, K)
    ew3 = edge_weight.reshape(NS, NCHUNK, K)
    x2 = jnp.concatenate([x[:, :HALF], x[:, HALF:]], axis=0)

    agg2, dis = _sc_aggregate(x2, row3, col3, ew3)
    agg = jnp.concatenate(
        [agg2[:N_NODES], agg2[NP:NP + N_NODES]], axis=1)
    dis2 = dis[:N_NODES].reshape(N_NODES, 1)

    return _tc_finish(x, agg, dis2, W, b.reshape(1, D_OUT))


# speedup vs baseline: 13.9062x; 13.9062x over previous
import jax
import jax.numpy as jnp
from jax import lax
from jax.experimental import pallas as pl
from jax.experimental.pallas import tpu as pltpu
from jax.experimental.pallas import tpu_sc as plsc

N_NODES = 10000
N_EDGES = 160000
D_IN = 256
D_OUT = 512
HALF = D_IN // 2

NC = 2
NS = 16
L = 16

EPT = N_EDGES // NS
K = 80
NCHUNK = EPT // K
NP = 10240
SLICE = NP // NS
ROWS = N_NODES // NS


def _rsqrt_pos(d):
    i = plsc.bitcast(d, jnp.int32)
    i = jnp.int32(0x5F3759DF) - lax.shift_right_logical(i, 1)
    y = plsc.bitcast(i, jnp.float32)
    half_d = 0.5 * d
    for _ in range(3):
        y = y * (1.5 - half_d * y * y)
    return y


def _sc_body(x2_hbm, row2_hbm, col3_hbm, ew3_hbm,
             agg_hbm, dis_hbm,
             agg_sp, deg_sh,
             col2d, ewn2d,
             deg_local, sbuf, rbuf, nbuf,
             gbuf, gsem):
    c = lax.axis_index("c")
    s = lax.axis_index("s")
    base = s * SLICE
    zero16 = jnp.zeros((L,), jnp.float32)
    iota16 = lax.iota(jnp.int32, L)

    pltpu.sync_copy(col3_hbm.at[s], col2d)
    pltpu.sync_copy(ew3_hbm.at[s], ewn2d)

    def zero_deg(i, _):
        deg_local[pl.ds(i * L, L)] = zero16
        return 0
    lax.fori_loop(0, NP // L, zero_deg, 0)

    def deg_acc(j, _):
        for k in range(K // L):
            c16 = col2d[j, pl.ds(k * L, L)]
            w16 = ewn2d[j, pl.ds(k * L, L)]
            plsc.addupdate_scatter(deg_local, [c16], w16)
        return 0
    lax.fori_loop(0, NCHUNK, deg_acc, 0)

    def zero_g(e, _):
        for q in range(HALF // L):
            gbuf[e, pl.ds(q * L, L)] = zero16
        return 0
    lax.fori_loop(0, K, zero_g, 0)

    def zero_agg(t, _):
        pltpu.sync_copy(gbuf.at[pl.ds(0, 25)],
                        agg_sp.at[pl.ds(s * ROWS + t * 25, 25)])
        return 0
    lax.fori_loop(0, ROWS // 25, zero_agg, 0)

    def zero_s(i, _):
        sbuf[pl.ds(i * L, L)] = zero16
        return 0
    lax.fori_loop(0, SLICE // L, zero_s, 0)
    pltpu.sync_copy(sbuf, deg_sh.at[pl.ds(base, SLICE)])

    plsc.subcore_barrier()

    def pub_deg(t, _):
        for q in range(K // L):
            rbuf[pl.ds(q * L, L)] = iota16 + (t * K + q * L)
        pltpu.sync_copy(deg_local.at[pl.ds(t * K, K)],
                        deg_sh.at[rbuf], add=True)
        return 0
    lax.fori_loop(0, N_NODES // K, pub_deg, 0)
    plsc.subcore_barrier()

    pltpu.sync_copy(deg_sh.at[pl.ds(base, SLICE)], sbuf)

    def calc_dis(i, _):
        d = sbuf[pl.ds(i * L, L)] + 1.0
        sbuf[pl.ds(i * L, L)] = _rsqrt_pos(d)
        return 0
    lax.fori_loop(0, SLICE // L, calc_dis, 0)

    pltpu.sync_copy(sbuf, deg_sh.at[pl.ds(base, SLICE)])

    @pl.when(c == 0)
    def _():
        pltpu.sync_copy(sbuf, dis_hbm.at[pl.ds(base, SLICE)])

    plsc.subcore_barrier()
    pltpu.sync_copy(deg_sh, deg_local)

    cN = c * N_NODES
    ebase = s * NCHUNK

    def chunk(j, _):
        pltpu.sync_copy(row2_hbm.at[ebase + j], rbuf)

        def norm_grp(g, _):
            sl = pl.ds(g * L, L)
            r16 = rbuf[sl]
            c16 = col2d[j, sl]
            w16 = ewn2d[j, sl]
            dr = plsc.load_gather(deg_local, [r16])
            dc = plsc.load_gather(deg_local, [c16])
            nbuf[sl] = dr * w16 * dc
            rbuf[sl] = r16 + cN
            return 0
        lax.fori_loop(0, K // L, norm_grp, 0)

        pltpu.async_copy(x2_hbm.at[rbuf], gbuf, gsem).wait()

        def scale(g, _):
            nv = nbuf[pl.ds(g * L, L)]
            for t in range(L):
                sv = nv[t]
                e = g * L + t
                for q in range(HALF // L):
                    sl = pl.ds(q * L, L)
                    gbuf[e, sl] = gbuf[e, sl] * sv
            return 0
        lax.fori_loop(0, K // L, scale, 0)

        pltpu.sync_copy(gbuf, agg_sp.at[col2d.at[j]], add=True)
        return 0
    lax.fori_loop(0, NCHUNK, chunk, 0)

    plsc.subcore_barrier()
    pltpu.sync_copy(agg_sp.at[pl.ds(s * ROWS, ROWS)],
                    agg_hbm.at[pl.ds(c * N_NODES + s * ROWS, ROWS)])


def _sc_aggregate(x2, row2, col3, ew3):
    mesh = plsc.VectorSubcoreMesh(core_axis_name="c", subcore_axis_name="s",
                                  num_cores=NC, num_subcores=NS)
    return pl.kernel(
        _sc_body,
        out_type=(
            jax.ShapeDtypeStruct((NC * N_NODES, HALF), jnp.float32),
            jax.ShapeDtypeStruct((NP,), jnp.float32),
        ),
        mesh=mesh,
        compiler_params=pltpu.CompilerParams(needs_layout_passes=False,
                                             use_tc_tiling_on_sc=False),
        scratch_types=[
            pltpu.VMEM_SHARED((N_NODES, HALF), jnp.float32),
            pltpu.VMEM_SHARED((NP,), jnp.float32),
            pltpu.VMEM((NCHUNK, K), jnp.int32),
            pltpu.VMEM((NCHUNK, K), jnp.float32),
            pltpu.VMEM((NP,), jnp.float32),
            pltpu.VMEM((SLICE,), jnp.float32),
            pltpu.VMEM((K,), jnp.int32),
            pltpu.VMEM((K,), jnp.float32),
            pltpu.VMEM((K, HALF), jnp.float32),
            pltpu.SemaphoreType.DMA,
        ],
    )(x2, row2, col3, ew3)


_RBLK = 1000


def _tc_body(x_ref, agg_ref, dis_ref, w_ref, b_ref, out_ref):
    d = dis_ref[...]
    a = agg_ref[...] + (d * d) * x_ref[...]
    acc = jnp.dot(a, w_ref[...], preferred_element_type=jnp.float32)
    out_ref[...] = jnp.maximum(acc + b_ref[...], 0.0)


def _tc_finish(x, agg, dis2, W, b2):
    grid = (N_NODES // _RBLK,)
    return pl.pallas_call(
        _tc_body,
        grid=grid,
        in_specs=[
            pl.BlockSpec((_RBLK, D_IN), lambda i: (i, 0)),
            pl.BlockSpec((_RBLK, D_IN), lambda i: (i, 0)),
            pl.BlockSpec((_RBLK, 1), lambda i: (i, 0)),
            pl.BlockSpec((D_IN, D_OUT), lambda i: (0, 0)),
            pl.BlockSpec((1, D_OUT), lambda i: (0, 0)),
        ],
        out_specs=pl.BlockSpec((_RBLK, D_OUT), lambda i: (i, 0)),
        out_shape=jax.ShapeDtypeStruct((N_NODES, D_OUT), jnp.float32),
    )(x, agg, dis2, W, b2)


def kernel(x, edge_index, edge_weight, W, b):
    row = edge_index[0].astype(jnp.int32)
    col = edge_index[1].astype(jnp.int32)
    row2 = row.reshape(NS * NCHUNK, K)
    col3 = col.reshape(NS, NCHUNK, K)
    ew3 = edge_weight.reshape(NS, NCHUNK, K)
    x2 = jnp.concatenate([x[:, :HALF], x[:, HALF:]], axis=0)

    agg2, dis = _sc_aggregate(x2, row2, col3, ew3)
    agg = jnp.concatenate(
        [agg2[:N_NODES], agg2[N_NODES:]], axis=1)
    dis2 = dis[:N_NODES].reshape(N_NODES, 1)

    return _tc_finish(x, agg, dis2, W, b.reshape(1, D_OUT))


# speedup vs baseline: 23.2091x; 1.6690x over previous
import jax
import jax.numpy as jnp
from jax import lax
from jax.experimental import pallas as pl
from jax.experimental.pallas import tpu as pltpu
from jax.experimental.pallas import tpu_sc as plsc

N_NODES = 10000
N_EDGES = 160000
D_IN = 256
D_OUT = 512
HALF = D_IN // 2

NC = 2
NS = 16
L = 16

EPT = N_EDGES // NS
K = 80
NCHUNK = EPT // K
NP = 10240
SLICE = NP // NS
ROWS = N_NODES // NS

_SC_PARAMS = pltpu.CompilerParams(needs_layout_passes=False,
                                  use_tc_tiling_on_sc=False)


def _rsqrt_pos(d):
    i = plsc.bitcast(d, jnp.int32)
    i = jnp.int32(0x5F3759DF) - lax.shift_right_logical(i, 1)
    y = plsc.bitcast(i, jnp.float32)
    half_d = 0.5 * d
    for _ in range(3):
        y = y * (1.5 - half_d * y * y)
    return y


def _degdis_body(col1_hbm, ew1_hbm, dis_hbm,
                 deg_sh, col1d, ew1d, deg_local, sbuf, rbuf):
    c = lax.axis_index("c")
    s = lax.axis_index("s")
    base = s * SLICE
    zero16 = jnp.zeros((L,), jnp.float32)
    iota16 = lax.iota(jnp.int32, L)

    @pl.when(c == 0)
    def _():
        pltpu.sync_copy(col1_hbm.at[s], col1d)
        pltpu.sync_copy(ew1_hbm.at[s], ew1d)

        def zero_deg(i, _):
            deg_local[pl.ds(i * L, L)] = zero16
            return 0
        lax.fori_loop(0, N_NODES // L, zero_deg, 0)

        def deg_acc(g, _):
            c16 = col1d[pl.ds(g * L, L)]
            w16 = ew1d[pl.ds(g * L, L)]
            plsc.addupdate_scatter(deg_local, [c16], w16)
            return 0
        lax.fori_loop(0, EPT // L, deg_acc, 0)

        def zero_s(i, _):
            sbuf[pl.ds(i * L, L)] = zero16
            return 0
        lax.fori_loop(0, SLICE // L, zero_s, 0)
        pltpu.sync_copy(sbuf, deg_sh.at[pl.ds(base, SLICE)])

        plsc.subcore_barrier()

        def pub_deg(t, _):
            for q in range(K // L):
                rbuf[pl.ds(q * L, L)] = iota16 + (t * K + q * L)
            pltpu.sync_copy(deg_local.at[pl.ds(t * K, K)],
                            deg_sh.at[rbuf], add=True)
            return 0
        lax.fori_loop(0, N_NODES // K, pub_deg, 0)
        plsc.subcore_barrier()

        pltpu.sync_copy(deg_sh.at[pl.ds(base, SLICE)], sbuf)

        def calc_dis(i, _):
            d = sbuf[pl.ds(i * L, L)] + 1.0
            sbuf[pl.ds(i * L, L)] = _rsqrt_pos(d)
            return 0
        lax.fori_loop(0, SLICE // L, calc_dis, 0)

        pltpu.sync_copy(sbuf, dis_hbm.at[pl.ds(base, SLICE)])


def _sc_degdis(col1, ew1):
    mesh = plsc.VectorSubcoreMesh(core_axis_name="c", subcore_axis_name="s",
                                  num_cores=NC, num_subcores=NS)
    return pl.kernel(
        _degdis_body,
        out_type=jax.ShapeDtypeStruct((NP,), jnp.float32),
        mesh=mesh,
        compiler_params=_SC_PARAMS,
        scratch_types=[
            pltpu.VMEM_SHARED((NP,), jnp.float32),
            pltpu.VMEM((EPT,), jnp.int32),
            pltpu.VMEM((EPT,), jnp.float32),
            pltpu.VMEM((N_NODES,), jnp.float32),
            pltpu.VMEM((SLICE,), jnp.float32),
            pltpu.VMEM((K,), jnp.int32),
        ],
    )(col1, ew1)


def _gather_body(y2_hbm, row1_hbm, col3_hbm, ew1_hbm,
                 agg_hbm,
                 agg_sp, row1d, col2d, ew1d, gbuf2, gsem0, gsem1):
    c = lax.axis_index("c")
    s = lax.axis_index("s")
    zero16 = jnp.zeros((L,), jnp.float32)
    cN = c * N_NODES

    pltpu.sync_copy(row1_hbm.at[s], row1d)
    pltpu.sync_copy(col3_hbm.at[s], col2d)
    pltpu.sync_copy(ew1_hbm.at[s], ew1d)

    def zero_g(e, _):
        for q in range(HALF // L):
            gbuf2[0, e, pl.ds(q * L, L)] = zero16
        return 0
    lax.fori_loop(0, K, zero_g, 0)

    def zero_agg(t, _):
        pltpu.sync_copy(gbuf2.at[0].at[pl.ds(0, 25)],
                        agg_sp.at[pl.ds(s * ROWS + t * 25, 25)])
        return 0
    lax.fori_loop(0, ROWS // 25, zero_agg, 0)

    def offs(g, _):
        sl = pl.ds(g * L, L)
        row1d[sl] = row1d[sl] + cN
        return 0
    lax.fori_loop(0, EPT // L, offs, 0)

    plsc.subcore_barrier()

    sems = (gsem0, gsem1)

    def fire(jn, p):
        pltpu.async_copy(y2_hbm.at[row1d.at[pl.ds(jn * K, K)]],
                         gbuf2.at[p], sems[p])

    def consume(j, p):
        pltpu.make_async_copy(y2_hbm.at[row1d.at[pl.ds(j * K, K)]],
                              gbuf2.at[p], sems[p]).wait()

        def scale(g, _):
            nv = ew1d[pl.ds(j * K + g * L, L)]
            for t in range(L):
                sv = nv[t]
                e = g * L + t
                for q in range(HALF // L):
                    sl = pl.ds(q * L, L)
                    gbuf2[p, e, sl] = gbuf2[p, e, sl] * sv
            return 0
        lax.fori_loop(0, K // L, scale, 0)

        pltpu.sync_copy(gbuf2.at[p], agg_sp.at[col2d.at[j]], add=True)

    fire(0, 0)
    fire(1, 1)

    def pair(jj, _):
        j0 = 2 * jj
        consume(j0, 0)
        fire(j0 + 2, 0)

        @pl.when(jj < NCHUNK // 2 - 1)
        def _():
            consume(j0 + 1, 1)
            fire(j0 + 3, 1)

        @pl.when(jj == NCHUNK // 2 - 1)
        def _():
            consume(j0 + 1, 1)
        return 0
    lax.fori_loop(0, NCHUNK // 2, pair, 0)
    consume(NCHUNK - 1, 0)

    plsc.subcore_barrier()
    pltpu.sync_copy(agg_sp.at[pl.ds(s * ROWS, ROWS)],
                    agg_hbm.at[pl.ds(c * N_NODES + s * ROWS, ROWS)])


def _sc_gather(y2, row1, col3, ew1):
    mesh = plsc.VectorSubcoreMesh(core_axis_name="c", subcore_axis_name="s",
                                  num_cores=NC, num_subcores=NS)
    return pl.kernel(
        _gather_body,
        out_type=jax.ShapeDtypeStruct((NC * N_NODES, HALF), jnp.float32),
        mesh=mesh,
        compiler_params=_SC_PARAMS,
        scratch_types=[
            pltpu.VMEM_SHARED((N_NODES, HALF), jnp.float32),
            pltpu.VMEM((EPT,), jnp.int32),
            pltpu.VMEM((NCHUNK, K), jnp.int32),
            pltpu.VMEM((EPT,), jnp.float32),
            pltpu.VMEM((2, K, HALF), jnp.float32),
            pltpu.SemaphoreType.DMA,
            pltpu.SemaphoreType.DMA,
        ],
    )(y2, row1, col3, ew1)


def _scale_body(x2_ref, dis_ref, out_ref):
    out_ref[...] = x2_ref[...] * dis_ref[...]


def _tc_scale_y(x2, disx):
    blk = 2000
    return pl.pallas_call(
        _scale_body,
        grid=(NC * N_NODES // blk,),
        in_specs=[
            pl.BlockSpec((blk, HALF), lambda i: (i, 0)),
            pl.BlockSpec((blk, 1), lambda i: (i, 0)),
        ],
        out_specs=pl.BlockSpec((blk, HALF), lambda i: (i, 0)),
        out_shape=jax.ShapeDtypeStruct((NC * N_NODES, HALF), jnp.float32),
    )(x2, disx)


_RBLK = 1000


def _tc_body(x_ref, agg_ref, dis_ref, w_ref, b_ref, out_ref):
    d = dis_ref[...]
    a = d * agg_ref[...] + (d * d) * x_ref[...]
    acc = jnp.dot(a, w_ref[...], preferred_element_type=jnp.float32)
    out_ref[...] = jnp.maximum(acc + b_ref[...], 0.0)


def _tc_finish(x, agg, dis2, W, b2):
    return pl.pallas_call(
        _tc_body,
        grid=(N_NODES // _RBLK,),
        in_specs=[
            pl.BlockSpec((_RBLK, D_IN), lambda i: (i, 0)),
            pl.BlockSpec((_RBLK, D_IN), lambda i: (i, 0)),
            pl.BlockSpec((_RBLK, 1), lambda i: (i, 0)),
            pl.BlockSpec((D_IN, D_OUT), lambda i: (0, 0)),
            pl.BlockSpec((1, D_OUT), lambda i: (0, 0)),
        ],
        out_specs=pl.BlockSpec((_RBLK, D_OUT), lambda i: (i, 0)),
        out_shape=jax.ShapeDtypeStruct((N_NODES, D_OUT), jnp.float32),
    )(x, agg, dis2, W, b2)


def kernel(x, edge_index, edge_weight, W, b):
    row = edge_index[0].astype(jnp.int32)
    col = edge_index[1].astype(jnp.int32)
    row1 = row.reshape(NS, EPT)
    col1 = col.reshape(NS, EPT)
    col3 = col.reshape(NS, NCHUNK, K)
    ew1 = edge_weight.reshape(NS, EPT)
    x2 = jnp.concatenate([x[:, :HALF], x[:, HALF:]], axis=0)

    dis = _sc_degdis(col1, ew1)
    disn = dis[:N_NODES].reshape(N_NODES, 1)
    disx = jnp.concatenate([disn, disn], axis=0)
    y2 = _tc_scale_y(x2, disx)
    agg2 = _sc_gather(y2, row1, col3, ew1)
    agg = jnp.concatenate([agg2[:N_NODES], agg2[N_NODES:]], axis=1)

    return _tc_finish(x, agg, disn, W, b.reshape(1, D_OUT))
